# Initial kernel scaffold; baseline (speedup 1.0000x reference)
import jax
import jax.numpy as jnp
from jax.experimental import pallas as pl


def kernel(particle_pt, particle_eta, particle_phi, particle_dep_energy, pt_eta_phi_pred, class_pred, energy_l_0, energy_l_1, particle_class, particle_idx, edge_src, edge_dst, parent_target, isIso):
    raise NotImplementedError("write your pallas kernel here")


# speedup vs baseline: 185.3937x; 185.3937x over previous
import functools

import jax
import jax.numpy as jnp
from jax import lax
from jax.experimental import pallas as pl
from jax.experimental.pallas import tpu as pltpu
from jax.experimental.pallas import tpu_sc as plsc

L = 16
NW = 32
EBLK = 4096
GCH = 128


def _sc_loss_kernel(N_P, N_F, E, SUBW,
                    ppt_hbm, cpred_hbm, pp_hbm, e0_hbm, e1_hbm, pcls_hbm,
                    esrc_hbm, edst_hbm, ptgt_hbm, iso_hbm, offs_hbm,
                    out_hbm,
                    offs_v, pt_sl, iso_sl, e0_sl, e1_sl, cp_sl, pp_sl,
                    cls_g, ppt_g, cnt, ebs0, ebs1, ebd0, ebd1, sh_i, sh_f, acc_v,
                    sem_pre, sem_e0, sem_e1):
    NV = SUBW // L
    wid = lax.axis_index("c") * 16 + lax.axis_index("s")
    own_lo = wid * SUBW
    slice_start = pl.multiple_of(jnp.minimum(own_lo, N_F - SUBW), 16)
    own_hi = jnp.minimum(own_lo + SUBW, N_F)

    pltpu.sync_copy(offs_hbm, offs_v)
    pltpu.sync_copy(ptgt_hbm.at[pl.ds(slice_start, SUBW)], pt_sl)
    iota = lax.iota(jnp.int32, L)

    def offs_at(i):
        d0 = jnp.sum(jnp.where(iota == i, offs_v[pl.ds(0, L)], 0))
        d1 = jnp.sum(jnp.where(iota == i - L, offs_v[pl.ds(L, L)], 0))
        d2 = jnp.sum(jnp.where(iota == i - 2 * L, offs_v[pl.ds(2 * L, L)], 0))
        return jnp.where(i < L, d0, jnp.where(i < 2 * L, d1, d2))

    pre = []
    pre.append(pltpu.async_copy(iso_hbm.at[pl.ds(slice_start, SUBW)],
                                iso_sl, sem_pre))
    pre.append(pltpu.async_copy(e0_hbm.at[pl.ds(slice_start, SUBW)],
                                e0_sl, sem_pre))
    pre.append(pltpu.async_copy(e1_hbm.at[pl.ds(slice_start, SUBW)],
                                e1_sl, sem_pre))
    pre.append(pltpu.async_copy(cpred_hbm.at[pl.ds(4 * slice_start, 4 * SUBW)],
                                cp_sl, sem_pre))
    pre.append(pltpu.async_copy(pp_hbm.at[pl.ds(3 * slice_start, 3 * SUBW)],
                                pp_sl, sem_pre))
    for j in range(0, SUBW, GCH):
        w = min(GCH, SUBW - j)
        sl = pl.ds(j, w)
        pre.append(pltpu.async_copy(pcls_hbm.at[pt_sl.at[sl]],
                                    cls_g.at[sl], sem_pre))
        pre.append(pltpu.async_copy(ppt_hbm.at[pt_sl.at[sl]],
                                    ppt_g.at[sl], sem_pre))

    zv = jnp.zeros((L,), jnp.float32)

    @pl.loop(0, NV)
    def _(v):
        cnt[pl.ds(v * L, L)] = zv

    sh_i[pl.ds(0, L)] = jnp.full((L,), -1, jnp.int32)
    sh_i[pl.ds(L, L)] = jnp.full((L,), -1, jnp.int32)
    sh_f[pl.ds(0, L)] = zv

    s = offs_at(wid)
    e = offs_at(wid + 1)
    s_al = jnp.bitwise_and(s, jnp.int32(-8))
    nblk = (e - s_al + (EBLK - 1)) // EBLK

    def blk_start(k):
        return pl.multiple_of(jnp.minimum(s_al + k * EBLK, E - EBLK), 8)

    ebs = (ebs0, ebs1)
    ebd = (ebd0, ebd1)
    sems = (sem_e0, sem_e1)

    def issue(k, b):
        st = blk_start(k)
        pltpu.async_copy(esrc_hbm.at[pl.ds(st, EBLK)], ebs[b], sems[b])
        pltpu.async_copy(edst_hbm.at[pl.ds(st, EBLK)], ebd[b], sems[b])

    def wait(k, b):
        st = blk_start(k)
        pltpu.make_async_copy(esrc_hbm.at[pl.ds(st, EBLK)], ebs[b], sems[b]).wait()
        pltpu.make_async_copy(edst_hbm.at[pl.ds(st, EBLK)], ebd[b], sems[b]).wait()

    def process(k, b):
        st = blk_start(k)
        lo_v = jnp.maximum(s, s_al + k * EBLK)

        @pl.loop(0, EBLK // L)
        def _(v):
            d = ebd[b][pl.ds(v * L, L)]
            sv = ebs[b][pl.ds(v * L, L)]
            eidx = st + v * L + iota
            valid = (eidx >= lo_v) & (eidx < e)
            li = d - slice_start
            li_c = jnp.minimum(jnp.maximum(li, 0), SUBW - 1)
            ptv = plsc.load_gather(pt_sl, [li_c])
            mf = jnp.where(valid & (sv == ptv), 1.0, 0.0)
            sh_i[pl.ds(1, L)] = d
            d_prev = sh_i[pl.ds(0, L)]
            d_next = sh_i[pl.ds(2, L)]
            is_first = d != d_prev
            is_last = d != d_next
            si = plsc.cummax(jnp.where(is_first, iota, 0))
            c = plsc.cumsum(mf)
            sh_f[pl.ds(1, L)] = c
            cprev = plsc.load_gather(sh_f, [si])
            tot = c - cprev
            plsc.addupdate_scatter(cnt, [li_c], tot,
                                   mask=is_last & (tot > 0.0))

    @pl.when(nblk > 0)
    def _():
        issue(0, 0)

    @pl.loop(0, nblk, step=2)
    def _(g):
        for b in (0, 1):
            k = g + b

            @pl.when(k < nblk)
            def _():
                @pl.when(k + 1 < nblk)
                def _():
                    issue(k + 1, 1 - b)

                wait(k, b)
                process(k, b)

    for h in pre:
        h.wait()

    def node_body(v, carry):
        acc_c, acc_p = carry
        lf = v * L + iota
        f = slice_start + lf
        validn = (f >= own_lo) & (f < own_hi)
        cv = jnp.where(validn, cnt[pl.ds(v * L, L)], 0.0)
        clsv = cls_g[pl.ds(v * L, L)].astype(jnp.float32)
        pcls = clsv * cv
        isov = iso_sl[pl.ds(v * L, L)].astype(jnp.float32) * cv
        em = e0_sl[pl.ds(v * L, L)] + e1_sl[pl.ds(v * L, L)]
        pcls = jnp.where(pcls == 2.0, 0.0, pcls)
        pcls = jnp.where((pcls == 3.0) & (isov == 1.0), 1.0, pcls)
        pcls = jnp.where((pcls == 3.0) & (isov == 0.0), 2.0, pcls)
        pcls = jnp.where(pcls == 4.0, 3.0, pcls)
        em0 = em == 0.0
        pcls = jnp.where(em0 & (pcls == 1.0), 0.0, pcls)
        pcls = jnp.where(em0 & (pcls == 2.0), 0.0, pcls)
        t = jnp.clip(pcls.astype(jnp.int32), 0, 3)

        i4 = 4 * (v * L) + 4 * iota
        l0 = plsc.load_gather(cp_sl, [i4])
        l1 = plsc.load_gather(cp_sl, [i4 + 1])
        l2 = plsc.load_gather(cp_sl, [i4 + 2])
        l3 = plsc.load_gather(cp_sl, [i4 + 3])
        m = jnp.maximum(jnp.maximum(l0, l1), jnp.maximum(l2, l3))
        ssum = (jnp.exp(l0 - m) + jnp.exp(l1 - m)
                + jnp.exp(l2 - m) + jnp.exp(l3 - m))
        tq = (ssum - 1.0) / (ssum + 1.0)
        y = tq * (2.0 + tq * tq * (2.0 / 3.0))
        y = y + ssum * jnp.exp(-y) - 1.0
        y = y + ssum * jnp.exp(-y) - 1.0
        lse = m + y
        lt = jnp.where(t == 0, l0, jnp.where(t == 1, l1,
                       jnp.where(t == 2, l2, l3)))
        wt = jnp.where(t == 0, 0.5, jnp.where(t == 1, 2.0,
                       jnp.where(t == 2, 5.0, 2.5)))
        acc_c = acc_c + jnp.where(validn, wt * (lse - lt), 0.0)
        pred0 = plsc.load_gather(pp_sl, [3 * (v * L) + 3 * iota])
        diff = pred0 - ppt_g[pl.ds(v * L, L)] * cv
        acc_p = acc_p + jnp.where(validn, diff * diff, 0.0)
        return acc_c, acc_p

    acc_c, acc_p = pl.loop(0, NV, init_carry=(zv, zv))(node_body)

    acc_v[pl.ds(0, L)] = acc_c
    acc_v[pl.ds(L, L)] = acc_p
    pltpu.sync_copy(acc_v, out_hbm.at[wid])


def kernel(particle_pt, particle_eta, particle_phi, particle_dep_energy,
           pt_eta_phi_pred, class_pred, energy_l_0, energy_l_1,
           particle_class, particle_idx, edge_src, edge_dst,
           parent_target, isIso):
    N_P = particle_pt.shape[0]
    N_F = parent_target.shape[0]
    E = edge_src.shape[0]
    SUBW = ((N_F + NW * L - 1) // (NW * L)) * L

    cpred_flat = class_pred.reshape(-1)
    pp_flat = pt_eta_phi_pred.reshape(-1)
    bounds = jnp.minimum(jnp.arange(NW + 1, dtype=jnp.int32) * SUBW, N_F)
    offs = jnp.searchsorted(edge_dst, bounds, side="left").astype(jnp.int32)
    offs = jnp.pad(offs, (0, 3 * L - (NW + 1)))

    mesh = plsc.VectorSubcoreMesh(core_axis_name="c", subcore_axis_name="s")
    sck = functools.partial(
        pl.kernel,
        out_type=jax.ShapeDtypeStruct((NW, 2 * L), jnp.float32),
        mesh=mesh,
        compiler_params=pltpu.CompilerParams(needs_layout_passes=False),
        scratch_types=[
            pltpu.VMEM((3 * L,), jnp.int32),
            pltpu.VMEM((SUBW,), jnp.int32),
            pltpu.VMEM((SUBW,), jnp.int32),
            pltpu.VMEM((SUBW,), jnp.float32),
            pltpu.VMEM((SUBW,), jnp.float32),
            pltpu.VMEM((4 * SUBW,), jnp.float32),
            pltpu.VMEM((3 * SUBW,), jnp.float32),
            pltpu.VMEM((SUBW,), jnp.int32),
            pltpu.VMEM((SUBW,), jnp.float32),
            pltpu.VMEM((SUBW,), jnp.float32),
            pltpu.VMEM((EBLK,), jnp.int32),
            pltpu.VMEM((EBLK,), jnp.int32),
            pltpu.VMEM((EBLK,), jnp.int32),
            pltpu.VMEM((EBLK,), jnp.int32),
            pltpu.VMEM((2 * L,), jnp.int32),
            pltpu.VMEM((2 * L,), jnp.float32),
            pltpu.VMEM((2 * L,), jnp.float32),
            pltpu.SemaphoreType.DMA,
            pltpu.SemaphoreType.DMA,
            pltpu.SemaphoreType.DMA,
        ],
    )(functools.partial(_sc_loss_kernel, N_P, N_F, E, SUBW))

    out = sck(particle_pt, cpred_flat, pp_flat, energy_l_0, energy_l_1,
              particle_class, edge_src, edge_dst, parent_target, isIso, offs)
    return (5.0 * jnp.sum(out[:, :L]) + jnp.sum(out[:, L:])) / N_F


# speedup vs baseline: 186.4377x; 1.0056x over previous
import functools

import jax
import jax.numpy as jnp
from jax import lax
from jax.experimental import pallas as pl
from jax.experimental.pallas import tpu as pltpu
from jax.experimental.pallas import tpu_sc as plsc

L = 16
NW = 32
EBLK = 4096
GCH = 128


def _sc_loss_kernel(N_P, N_F, E, SUBW,
                    ppt_hbm, cpred_hbm, pp_hbm, e0_hbm, e1_hbm, pcls_hbm,
                    esrc_hbm, edst_hbm, ptgt_hbm, iso_hbm, offs_hbm,
                    out_hbm,
                    offs_v, pt_sl, iso_sl, e0_sl, e1_sl, cp_sl, pp_sl,
                    cls_g, ppt_g, cnt, ebs0, ebs1, ebd0, ebd1, sh_i, sh_f, acc_v,
                    sem_pre, sem_e0, sem_e1):
    NV = SUBW // L
    wid = lax.axis_index("c") * 16 + lax.axis_index("s")
    own_lo = wid * SUBW
    slice_start = pl.multiple_of(jnp.minimum(own_lo, N_F - SUBW), 16)
    own_hi = jnp.minimum(own_lo + SUBW, N_F)

    pltpu.sync_copy(offs_hbm, offs_v)
    pltpu.sync_copy(ptgt_hbm.at[pl.ds(slice_start, SUBW)], pt_sl)
    iota = lax.iota(jnp.int32, L)

    def offs_at(i):
        d0 = jnp.sum(jnp.where(iota == i, offs_v[pl.ds(0, L)], 0))
        d1 = jnp.sum(jnp.where(iota == i - L, offs_v[pl.ds(L, L)], 0))
        d2 = jnp.sum(jnp.where(iota == i - 2 * L, offs_v[pl.ds(2 * L, L)], 0))
        return jnp.where(i < L, d0, jnp.where(i < 2 * L, d1, d2))

    pre = []
    pre.append(pltpu.async_copy(iso_hbm.at[pl.ds(slice_start, SUBW)],
                                iso_sl, sem_pre))
    pre.append(pltpu.async_copy(e0_hbm.at[pl.ds(slice_start, SUBW)],
                                e0_sl, sem_pre))
    pre.append(pltpu.async_copy(e1_hbm.at[pl.ds(slice_start, SUBW)],
                                e1_sl, sem_pre))
    pre.append(pltpu.async_copy(cpred_hbm.at[pl.ds(4 * slice_start, 4 * SUBW)],
                                cp_sl, sem_pre))
    pre.append(pltpu.async_copy(pp_hbm.at[pl.ds(slice_start, SUBW)],
                                pp_sl, sem_pre))
    for j in range(0, SUBW, GCH):
        w = min(GCH, SUBW - j)
        sl = pl.ds(j, w)
        pre.append(pltpu.async_copy(pcls_hbm.at[pt_sl.at[sl]],
                                    cls_g.at[sl], sem_pre))
        pre.append(pltpu.async_copy(ppt_hbm.at[pt_sl.at[sl]],
                                    ppt_g.at[sl], sem_pre))

    zv = jnp.zeros((L,), jnp.float32)

    @pl.loop(0, NV)
    def _(v):
        cnt[pl.ds(v * L, L)] = zv

    sh_i[pl.ds(0, L)] = jnp.full((L,), -1, jnp.int32)
    sh_i[pl.ds(L, L)] = jnp.full((L,), -1, jnp.int32)
    sh_f[pl.ds(0, L)] = zv

    s = offs_at(wid)
    e = offs_at(wid + 1)
    s_al = jnp.bitwise_and(s, jnp.int32(-8))
    nblk = (e - s_al + (EBLK - 1)) // EBLK

    def blk_start(k):
        return pl.multiple_of(jnp.minimum(s_al + k * EBLK, E - EBLK), 8)

    ebs = (ebs0, ebs1)
    ebd = (ebd0, ebd1)
    sems = (sem_e0, sem_e1)

    def issue(k, b):
        st = blk_start(k)
        pltpu.async_copy(esrc_hbm.at[pl.ds(st, EBLK)], ebs[b], sems[b])
        pltpu.async_copy(edst_hbm.at[pl.ds(st, EBLK)], ebd[b], sems[b])

    def wait(k, b):
        st = blk_start(k)
        pltpu.make_async_copy(esrc_hbm.at[pl.ds(st, EBLK)], ebs[b], sems[b]).wait()
        pltpu.make_async_copy(edst_hbm.at[pl.ds(st, EBLK)], ebd[b], sems[b]).wait()

    def process(k, b):
        st = blk_start(k)
        lo_v = jnp.maximum(s, s_al + k * EBLK)

        @pl.loop(0, EBLK // L, unroll=4)
        def _(v):
            d = ebd[b][pl.ds(v * L, L)]
            sv = ebs[b][pl.ds(v * L, L)]
            eidx = st + v * L + iota
            valid = (eidx >= lo_v) & (eidx < e)
            li = d - slice_start
            li_c = jnp.minimum(jnp.maximum(li, 0), SUBW - 1)
            ptv = plsc.load_gather(pt_sl, [li_c])
            match = valid & (sv == ptv)

            @pl.when(jnp.any(match))
            def _():
                mf = jnp.where(match, 1.0, 0.0)
                sh_i[pl.ds(1, L)] = d
                d_prev = sh_i[pl.ds(0, L)]
                d_next = sh_i[pl.ds(2, L)]
                is_first = d != d_prev
                is_last = d != d_next
                si = plsc.cummax(jnp.where(is_first, iota, 0))
                c = plsc.cumsum(mf)
                sh_f[pl.ds(1, L)] = c
                cprev = plsc.load_gather(sh_f, [si])
                tot = c - cprev
                plsc.addupdate_scatter(cnt, [li_c], tot,
                                       mask=is_last & (tot > 0.0))

    @pl.when(nblk > 0)
    def _():
        issue(0, 0)

    @pl.loop(0, nblk, step=2)
    def _(g):
        for b in (0, 1):
            k = g + b

            @pl.when(k < nblk)
            def _():
                @pl.when(k + 1 < nblk)
                def _():
                    issue(k + 1, 1 - b)

                wait(k, b)
                process(k, b)

    for h in pre:
        h.wait()

    def node_body(v, carry):
        acc_c, acc_p = carry
        lf = v * L + iota
        f = slice_start + lf
        validn = (f >= own_lo) & (f < own_hi)
        cv = jnp.where(validn, cnt[pl.ds(v * L, L)], 0.0)
        clsv = cls_g[pl.ds(v * L, L)].astype(jnp.float32)
        pcls = clsv * cv
        isov = iso_sl[pl.ds(v * L, L)].astype(jnp.float32) * cv
        em = e0_sl[pl.ds(v * L, L)] + e1_sl[pl.ds(v * L, L)]
        pcls = jnp.where(pcls == 2.0, 0.0, pcls)
        pcls = jnp.where((pcls == 3.0) & (isov == 1.0), 1.0, pcls)
        pcls = jnp.where((pcls == 3.0) & (isov == 0.0), 2.0, pcls)
        pcls = jnp.where(pcls == 4.0, 3.0, pcls)
        em0 = em == 0.0
        pcls = jnp.where(em0 & (pcls == 1.0), 0.0, pcls)
        pcls = jnp.where(em0 & (pcls == 2.0), 0.0, pcls)
        t = jnp.clip(pcls.astype(jnp.int32), 0, 3)

        i4 = 4 * (v * L) + 4 * iota
        l0 = plsc.load_gather(cp_sl, [i4])
        l1 = plsc.load_gather(cp_sl, [i4 + 1])
        l2 = plsc.load_gather(cp_sl, [i4 + 2])
        l3 = plsc.load_gather(cp_sl, [i4 + 3])
        m = jnp.maximum(jnp.maximum(l0, l1), jnp.maximum(l2, l3))
        ssum = (jnp.exp(l0 - m) + jnp.exp(l1 - m)
                + jnp.exp(l2 - m) + jnp.exp(l3 - m))
        tq = (ssum - 1.0) / (ssum + 1.0)
        y = tq * (2.0 + tq * tq * (2.0 / 3.0))
        y = y + ssum * jnp.exp(-y) - 1.0
        y = y + ssum * jnp.exp(-y) - 1.0
        lse = m + y
        lt = jnp.where(t == 0, l0, jnp.where(t == 1, l1,
                       jnp.where(t == 2, l2, l3)))
        wt = jnp.where(t == 0, 0.5, jnp.where(t == 1, 2.0,
                       jnp.where(t == 2, 5.0, 2.5)))
        acc_c = acc_c + jnp.where(validn, wt * (lse - lt), 0.0)
        pred0 = pp_sl[pl.ds(v * L, L)]
        diff = pred0 - ppt_g[pl.ds(v * L, L)] * cv
        acc_p = acc_p + jnp.where(validn, diff * diff, 0.0)
        return acc_c, acc_p

    acc_c, acc_p = pl.loop(0, NV, init_carry=(zv, zv), unroll=2)(node_body)

    acc_v[pl.ds(0, L)] = acc_c
    acc_v[pl.ds(L, L)] = acc_p
    pltpu.sync_copy(acc_v, out_hbm.at[wid])


def kernel(particle_pt, particle_eta, particle_phi, particle_dep_energy,
           pt_eta_phi_pred, class_pred, energy_l_0, energy_l_1,
           particle_class, particle_idx, edge_src, edge_dst,
           parent_target, isIso):
    N_P = particle_pt.shape[0]
    N_F = parent_target.shape[0]
    E = edge_src.shape[0]
    SUBW = ((N_F + NW * L - 1) // (NW * L)) * L

    cpred_flat = class_pred.reshape(-1)
    pp0 = pt_eta_phi_pred[:, 0]
    bounds = jnp.minimum(jnp.arange(NW + 1, dtype=jnp.int32) * SUBW, N_F)
    offs = jnp.searchsorted(edge_dst, bounds, side="left").astype(jnp.int32)
    offs = jnp.pad(offs, (0, 3 * L - (NW + 1)))

    mesh = plsc.VectorSubcoreMesh(core_axis_name="c", subcore_axis_name="s")
    sck = functools.partial(
        pl.kernel,
        out_type=jax.ShapeDtypeStruct((NW, 2 * L), jnp.float32),
        mesh=mesh,
        compiler_params=pltpu.CompilerParams(needs_layout_passes=False),
        scratch_types=[
            pltpu.VMEM((3 * L,), jnp.int32),
            pltpu.VMEM((SUBW,), jnp.int32),
            pltpu.VMEM((SUBW,), jnp.int32),
            pltpu.VMEM((SUBW,), jnp.float32),
            pltpu.VMEM((SUBW,), jnp.float32),
            pltpu.VMEM((4 * SUBW,), jnp.float32),
            pltpu.VMEM((SUBW,), jnp.float32),
            pltpu.VMEM((SUBW,), jnp.int32),
            pltpu.VMEM((SUBW,), jnp.float32),
            pltpu.VMEM((SUBW,), jnp.float32),
            pltpu.VMEM((EBLK,), jnp.int32),
            pltpu.VMEM((EBLK,), jnp.int32),
            pltpu.VMEM((EBLK,), jnp.int32),
            pltpu.VMEM((EBLK,), jnp.int32),
            pltpu.VMEM((2 * L,), jnp.int32),
            pltpu.VMEM((2 * L,), jnp.float32),
            pltpu.VMEM((2 * L,), jnp.float32),
            pltpu.SemaphoreType.DMA,
            pltpu.SemaphoreType.DMA,
            pltpu.SemaphoreType.DMA,
        ],
    )(functools.partial(_sc_loss_kernel, N_P, N_F, E, SUBW))

    out = sck(particle_pt, cpred_flat, pp0, energy_l_0, energy_l_1,
              particle_class, edge_src, edge_dst, parent_target, isIso, offs)
    return (5.0 * jnp.sum(out[:, :L]) + jnp.sum(out[:, L:])) / N_F


# speedup vs baseline: 222.9239x; 1.1957x over previous
import functools

import jax
import jax.numpy as jnp
from jax import lax
from jax.experimental import pallas as pl
from jax.experimental.pallas import tpu as pltpu
from jax.experimental.pallas import tpu_sc as plsc

L = 16
NW = 32
EBLK = 4096
GCH = 128


def _sc_loss_kernel(N_P, N_F, E, SUBW,
                    ppt_hbm, cpred_hbm, pp_hbm, e0_hbm, e1_hbm, pcls_hbm,
                    esrc_hbm, edst_hbm, ptgt_hbm, iso_hbm,
                    out_hbm,
                    pt_sl, iso_sl, e0_sl, e1_sl, cp_sl, pp_sl,
                    cls_g, ppt_g, cnt, ebs0, ebs1, ebd0, ebd1, sh_i, sh_f, acc_v,
                    pbiA, pbvA, pbiB, pbvB,
                    sem_pre, sem_e0, sem_e1, sem_sA, sem_sB):
    NV = SUBW // L
    wid = lax.axis_index("c") * 16 + lax.axis_index("s")
    own_lo = wid * SUBW
    slice_start = pl.multiple_of(jnp.minimum(own_lo, N_F - SUBW), 16)
    own_hi = jnp.minimum(own_lo + SUBW, N_F)

    pltpu.sync_copy(ptgt_hbm.at[pl.ds(slice_start, SUBW)], pt_sl)
    iota = lax.iota(jnp.int32, L)

    pre = []
    pre.append(pltpu.async_copy(iso_hbm.at[pl.ds(slice_start, SUBW)],
                                iso_sl, sem_pre))
    pre.append(pltpu.async_copy(e0_hbm.at[pl.ds(slice_start, SUBW)],
                                e0_sl, sem_pre))
    pre.append(pltpu.async_copy(e1_hbm.at[pl.ds(slice_start, SUBW)],
                                e1_sl, sem_pre))
    pre.append(pltpu.async_copy(cpred_hbm.at[pl.ds(4 * slice_start, 4 * SUBW)],
                                cp_sl, sem_pre))
    pre.append(pltpu.async_copy(pp_hbm.at[pl.ds(slice_start, SUBW)],
                                pp_sl, sem_pre))
    for j in range(0, SUBW, GCH):
        w = min(GCH, SUBW - j)
        sl = pl.ds(j, w)
        pre.append(pltpu.async_copy(pcls_hbm.at[pt_sl.at[sl]],
                                    cls_g.at[sl], sem_pre))
        pre.append(pltpu.async_copy(ppt_hbm.at[pt_sl.at[sl]],
                                    ppt_g.at[sl], sem_pre))

    zv = jnp.zeros((L,), jnp.float32)

    @pl.loop(0, NV)
    def _(v):
        cnt[pl.ds(v * L, L)] = zv

    sh_i[pl.ds(0, L)] = jnp.full((L,), -1, jnp.int32)
    sh_i[pl.ds(L, L)] = jnp.full((L,), -1, jnp.int32)
    sh_f[pl.ds(0, L)] = zv

    tA = own_lo
    tB = own_hi
    loA = jnp.int32(0)
    hiA = jnp.int32(E)
    loB = jnp.int32(0)
    hiB = jnp.int32(E)
    for _ in range(7):
        pA = jnp.minimum(loA + ((hiA - loA) * (iota + 1)) // 17, E - 1)
        pB = jnp.minimum(loB + ((hiB - loB) * (iota + 1)) // 17, E - 1)
        pbiA[...] = pA
        pbiB[...] = pB
        hA = pltpu.async_copy(edst_hbm.at[pbiA], pbvA, sem_sA)
        hB = pltpu.async_copy(edst_hbm.at[pbiB], pbvB, sem_sB)
        hA.wait()
        hB.wait()
        ltA = pbvA[...] < tA
        ltB = pbvB[...] < tB
        loA = jnp.max(jnp.where(ltA, pA + 1, loA))
        hiA = jnp.min(jnp.where(ltA, hiA, pA))
        loB = jnp.max(jnp.where(ltB, pB + 1, loB))
        hiB = jnp.min(jnp.where(ltB, hiB, pB))
    s = loA
    e = loB

    s_al = jnp.bitwise_and(s, jnp.int32(-8))
    nblk = (e - s_al + (EBLK - 1)) // EBLK

    def blk_start(k):
        return pl.multiple_of(jnp.minimum(s_al + k * EBLK, E - EBLK), 8)

    ebs = (ebs0, ebs1)
    ebd = (ebd0, ebd1)
    sems = (sem_e0, sem_e1)

    def issue(k, b):
        st = blk_start(k)
        pltpu.async_copy(esrc_hbm.at[pl.ds(st, EBLK)], ebs[b], sems[b])
        pltpu.async_copy(edst_hbm.at[pl.ds(st, EBLK)], ebd[b], sems[b])

    def wait(k, b):
        st = blk_start(k)
        pltpu.make_async_copy(esrc_hbm.at[pl.ds(st, EBLK)], ebs[b], sems[b]).wait()
        pltpu.make_async_copy(edst_hbm.at[pl.ds(st, EBLK)], ebd[b], sems[b]).wait()

    def process(k, b):
        st = blk_start(k)
        lo_v = jnp.maximum(s, s_al + k * EBLK)

        @pl.loop(0, EBLK // L, unroll=4)
        def _(v):
            d = ebd[b][pl.ds(v * L, L)]
            sv = ebs[b][pl.ds(v * L, L)]
            eidx = st + v * L + iota
            valid = (eidx >= lo_v) & (eidx < e)
            li = d - slice_start
            li_c = jnp.minimum(jnp.maximum(li, 0), SUBW - 1)
            ptv = plsc.load_gather(pt_sl, [li_c])
            match = valid & (sv == ptv)

            @pl.when(jnp.any(match))
            def _():
                mf = jnp.where(match, 1.0, 0.0)
                sh_i[pl.ds(1, L)] = d
                d_prev = sh_i[pl.ds(0, L)]
                d_next = sh_i[pl.ds(2, L)]
                is_first = d != d_prev
                is_last = d != d_next
                si = plsc.cummax(jnp.where(is_first, iota, 0))
                c = plsc.cumsum(mf)
                sh_f[pl.ds(1, L)] = c
                cprev = plsc.load_gather(sh_f, [si])
                tot = c - cprev
                plsc.addupdate_scatter(cnt, [li_c], tot,
                                       mask=is_last & (tot > 0.0))

    @pl.when(nblk > 0)
    def _():
        issue(0, 0)

    @pl.loop(0, nblk, step=2)
    def _(g):
        for b in (0, 1):
            k = g + b

            @pl.when(k < nblk)
            def _():
                @pl.when(k + 1 < nblk)
                def _():
                    issue(k + 1, 1 - b)

                wait(k, b)
                process(k, b)

    for h in pre:
        h.wait()

    def node_body(v, carry):
        acc_c, acc_p = carry
        lf = v * L + iota
        f = slice_start + lf
        validn = (f >= own_lo) & (f < own_hi)
        cv = jnp.where(validn, cnt[pl.ds(v * L, L)], 0.0)
        clsv = cls_g[pl.ds(v * L, L)].astype(jnp.float32)
        pcls = clsv * cv
        isov = iso_sl[pl.ds(v * L, L)].astype(jnp.float32) * cv
        em = e0_sl[pl.ds(v * L, L)] + e1_sl[pl.ds(v * L, L)]
        pcls = jnp.where(pcls == 2.0, 0.0, pcls)
        pcls = jnp.where((pcls == 3.0) & (isov == 1.0), 1.0, pcls)
        pcls = jnp.where((pcls == 3.0) & (isov == 0.0), 2.0, pcls)
        pcls = jnp.where(pcls == 4.0, 3.0, pcls)
        em0 = em == 0.0
        pcls = jnp.where(em0 & (pcls == 1.0), 0.0, pcls)
        pcls = jnp.where(em0 & (pcls == 2.0), 0.0, pcls)
        t = jnp.clip(pcls.astype(jnp.int32), 0, 3)

        i4 = 4 * (v * L) + 4 * iota
        l0 = plsc.load_gather(cp_sl, [i4])
        l1 = plsc.load_gather(cp_sl, [i4 + 1])
        l2 = plsc.load_gather(cp_sl, [i4 + 2])
        l3 = plsc.load_gather(cp_sl, [i4 + 3])
        m = jnp.maximum(jnp.maximum(l0, l1), jnp.maximum(l2, l3))
        ssum = (jnp.exp(l0 - m) + jnp.exp(l1 - m)
                + jnp.exp(l2 - m) + jnp.exp(l3 - m))
        tq = (ssum - 1.0) / (ssum + 1.0)
        y = tq * (2.0 + tq * tq * (2.0 / 3.0))
        y = y + ssum * jnp.exp(-y) - 1.0
        y = y + ssum * jnp.exp(-y) - 1.0
        lse = m + y
        lt = jnp.where(t == 0, l0, jnp.where(t == 1, l1,
                       jnp.where(t == 2, l2, l3)))
        wt = jnp.where(t == 0, 0.5, jnp.where(t == 1, 2.0,
                       jnp.where(t == 2, 5.0, 2.5)))
        acc_c = acc_c + jnp.where(validn, wt * (lse - lt), 0.0)
        pred0 = pp_sl[pl.ds(v * L, L)]
        diff = pred0 - ppt_g[pl.ds(v * L, L)] * cv
        acc_p = acc_p + jnp.where(validn, diff * diff, 0.0)
        return acc_c, acc_p

    acc_c, acc_p = pl.loop(0, NV, init_carry=(zv, zv), unroll=2)(node_body)

    acc_v[pl.ds(0, L)] = acc_c
    acc_v[pl.ds(L, L)] = acc_p
    pltpu.sync_copy(acc_v, out_hbm.at[wid])


def kernel(particle_pt, particle_eta, particle_phi, particle_dep_energy,
           pt_eta_phi_pred, class_pred, energy_l_0, energy_l_1,
           particle_class, particle_idx, edge_src, edge_dst,
           parent_target, isIso):
    N_P = particle_pt.shape[0]
    N_F = parent_target.shape[0]
    E = edge_src.shape[0]
    SUBW = ((N_F + NW * L - 1) // (NW * L)) * L

    cpred_flat = class_pred.reshape(-1)
    pp0 = pt_eta_phi_pred[:, 0]

    mesh = plsc.VectorSubcoreMesh(core_axis_name="c", subcore_axis_name="s")
    sck = functools.partial(
        pl.kernel,
        out_type=jax.ShapeDtypeStruct((NW, 2 * L), jnp.float32),
        mesh=mesh,
        compiler_params=pltpu.CompilerParams(needs_layout_passes=False),
        scratch_types=[
            pltpu.VMEM((SUBW,), jnp.int32),
            pltpu.VMEM((SUBW,), jnp.int32),
            pltpu.VMEM((SUBW,), jnp.float32),
            pltpu.VMEM((SUBW,), jnp.float32),
            pltpu.VMEM((4 * SUBW,), jnp.float32),
            pltpu.VMEM((SUBW,), jnp.float32),
            pltpu.VMEM((SUBW,), jnp.int32),
            pltpu.VMEM((SUBW,), jnp.float32),
            pltpu.VMEM((SUBW,), jnp.float32),
            pltpu.VMEM((EBLK,), jnp.int32),
            pltpu.VMEM((EBLK,), jnp.int32),
            pltpu.VMEM((EBLK,), jnp.int32),
            pltpu.VMEM((EBLK,), jnp.int32),
            pltpu.VMEM((2 * L,), jnp.int32),
            pltpu.VMEM((2 * L,), jnp.float32),
            pltpu.VMEM((2 * L,), jnp.float32),
            pltpu.VMEM((L,), jnp.int32),
            pltpu.VMEM((L,), jnp.int32),
            pltpu.VMEM((L,), jnp.int32),
            pltpu.VMEM((L,), jnp.int32),
            pltpu.SemaphoreType.DMA,
            pltpu.SemaphoreType.DMA,
            pltpu.SemaphoreType.DMA,
            pltpu.SemaphoreType.DMA,
            pltpu.SemaphoreType.DMA,
        ],
    )(functools.partial(_sc_loss_kernel, N_P, N_F, E, SUBW))

    out = sck(particle_pt, cpred_flat, pp0, energy_l_0, energy_l_1,
              particle_class, edge_src, edge_dst, parent_target, isIso)
    return (5.0 * jnp.sum(out[:, :L]) + jnp.sum(out[:, L:])) / N_F


# speedup vs baseline: 588.6712x; 2.6407x over previous
import functools

import jax
import jax.numpy as jnp
from jax import lax
from jax.experimental import pallas as pl
from jax.experimental.pallas import tpu as pltpu
from jax.experimental.pallas import tpu_sc as plsc

L = 16
NW = 32
EBLK = 4096


def _count_kernel(N_P, N_F, E, SUBW,
                  ppt_hbm, e0_hbm, e1_hbm, pcls_hbm,
                  esrc_hbm, edst_hbm, ptgt_hbm, iso_hbm,
                  outt_hbm, outpm_hbm,
                  pt_sl, iso_sl, e0_sl, e1_sl, gbuf_c, gbuf_p, cnt,
                  ebs0, ebs1, ebd0, ebd1, sh_i, sh_f, tb_sl, pm_sl,
                  pbiA, pbvA, pbiB, pbvB,
                  sem_pre, sem_e0, sem_e1, sem_sA, sem_sB, sem_g):
    NV = SUBW // L
    wid = lax.axis_index("c") * 16 + lax.axis_index("s")
    own_lo = wid * SUBW
    slice_start = pl.multiple_of(jnp.minimum(own_lo, N_F - SUBW), 16)
    own_hi = jnp.minimum(own_lo + SUBW, N_F)

    h_pt = pltpu.async_copy(ptgt_hbm.at[pl.ds(slice_start, SUBW)], pt_sl,
                            sem_pre)
    iota = lax.iota(jnp.int32, L)

    PB = 8 * L
    tA = own_lo
    tB = own_hi
    loA = jnp.int32(0)
    hiA = jnp.int32(E)
    loB = jnp.int32(0)
    hiB = jnp.int32(E)
    with jax.named_scope("edge_range_search"):
        for _ in range(4):
            spanA = hiA - loA
            spanB = hiB - loB
            for j in range(PB // L):
                k = j * L + iota + 1
                pbiA[pl.ds(j * L, L)] = jnp.minimum(
                    loA + (spanA * k) // (PB + 1), E - 1)
                pbiB[pl.ds(j * L, L)] = jnp.minimum(
                    loB + (spanB * k) // (PB + 1), E - 1)
            hA = pltpu.async_copy(edst_hbm.at[pbiA], pbvA, sem_sA)
            hB = pltpu.async_copy(edst_hbm.at[pbiB], pbvB, sem_sB)
            hA.wait()
            hB.wait()
            for j in range(PB // L):
                slj = pl.ds(j * L, L)
                pA = pbiA[slj]
                pB_ = pbiB[slj]
                ltA = pbvA[slj] < tA
                ltB = pbvB[slj] < tB
                loA = jnp.maximum(loA, jnp.max(jnp.where(ltA, pA + 1, 0)))
                hiA = jnp.minimum(hiA, jnp.min(jnp.where(ltA, hiA, pA)))
                loB = jnp.maximum(loB, jnp.max(jnp.where(ltB, pB_ + 1, 0)))
                hiB = jnp.minimum(hiB, jnp.min(jnp.where(ltB, hiB, pB_)))
    s = loA
    e = loB
    h_pt.wait()

    pre = []
    pre.append(pltpu.async_copy(iso_hbm.at[pl.ds(slice_start, SUBW)],
                                iso_sl, sem_pre))
    pre.append(pltpu.async_copy(e0_hbm.at[pl.ds(slice_start, SUBW)],
                                e0_sl, sem_pre))
    pre.append(pltpu.async_copy(e1_hbm.at[pl.ds(slice_start, SUBW)],
                                e1_sl, sem_pre))

    zv = jnp.zeros((L,), jnp.float32)

    @pl.loop(0, NV)
    def _(v):
        cnt[pl.ds(v * L, L)] = zv

    sh_i[pl.ds(0, L)] = jnp.full((L,), -1, jnp.int32)
    sh_i[pl.ds(L, L)] = jnp.full((L,), -1, jnp.int32)
    sh_f[pl.ds(0, L)] = zv

    s_al = jnp.bitwise_and(s, jnp.int32(-8))
    nblk = (e - s_al + (EBLK - 1)) // EBLK

    def blk_start(k):
        return pl.multiple_of(jnp.minimum(s_al + k * EBLK, E - EBLK), 8)

    ebs = (ebs0, ebs1)
    ebd = (ebd0, ebd1)
    sems = (sem_e0, sem_e1)

    def issue(k, b):
        st = blk_start(k)
        pltpu.async_copy(esrc_hbm.at[pl.ds(st, EBLK)], ebs[b], sems[b])
        pltpu.async_copy(edst_hbm.at[pl.ds(st, EBLK)], ebd[b], sems[b])

    def wait(k, b):
        st = blk_start(k)
        pltpu.make_async_copy(esrc_hbm.at[pl.ds(st, EBLK)], ebs[b], sems[b]).wait()
        pltpu.make_async_copy(edst_hbm.at[pl.ds(st, EBLK)], ebd[b], sems[b]).wait()

    def process(k, b):
        st = blk_start(k)
        lo_v = jnp.maximum(s, s_al + k * EBLK)

        def scan_vec(v, anyacc):
            d = ebd[b][pl.ds(v * L, L)]
            sv = ebs[b][pl.ds(v * L, L)]
            eidx = st + v * L + iota
            valid = (eidx >= lo_v) & (eidx < e)
            li = d - slice_start
            li_c = jnp.minimum(jnp.maximum(li, 0), SUBW - 1)
            ptv = plsc.load_gather(pt_sl, [li_c])
            match = valid & (sv == ptv)
            return anyacc | jnp.where(match, 1, 0)

        anyacc = pl.loop(0, EBLK // L,
                         init_carry=jnp.zeros((L,), jnp.int32),
                         unroll=4)(scan_vec)

        @pl.when(jnp.max(anyacc) > 0)
        def _():
            @pl.loop(0, EBLK // L)
            def _(v):
                d = ebd[b][pl.ds(v * L, L)]
                sv = ebs[b][pl.ds(v * L, L)]
                eidx = st + v * L + iota
                valid = (eidx >= lo_v) & (eidx < e)
                li = d - slice_start
                li_c = jnp.minimum(jnp.maximum(li, 0), SUBW - 1)
                ptv = plsc.load_gather(pt_sl, [li_c])
                match = valid & (sv == ptv)
                mf = jnp.where(match, 1.0, 0.0)
                sh_i[pl.ds(1, L)] = d
                d_prev = sh_i[pl.ds(0, L)]
                d_next = sh_i[pl.ds(2, L)]
                is_first = d != d_prev
                is_last = d != d_next
                si = plsc.cummax(jnp.where(is_first, iota, 0))
                c = plsc.cumsum(mf)
                sh_f[pl.ds(1, L)] = c
                cprev = plsc.load_gather(sh_f, [si])
                tot = c - cprev
                plsc.addupdate_scatter(cnt, [li_c], tot,
                                       mask=is_last & (tot > 0.0))

    @pl.when(nblk > 0)
    def _():
        issue(0, 0)

    with jax.named_scope("phase1_edges"):
        @pl.loop(0, nblk, step=2)
        def _(g):
            for b in (0, 1):
                k = g + b

                @pl.when(k < nblk)
                def _():
                    @pl.when(k + 1 < nblk)
                    def _():
                        issue(k + 1, 1 - b)

                    wait(k, b)
                    process(k, b)

    with jax.named_scope("drain_pre"):
        for h in pre:
            h.wait()

    with jax.named_scope("phase2_remap"):
        @pl.loop(0, NV, unroll=2)
        def _(v):
            lf = v * L + iota
            f = slice_start + lf
            validn = (f >= own_lo) & (f < own_hi)
            cv = jnp.where(validn, cnt[pl.ds(v * L, L)], 0.0)
            pos = cv > 0.0

            @pl.when(jnp.any(pos))
            def _():
                idxsl = pt_sl.at[pl.ds(pl.multiple_of(v * L, L), L)]
                hc = pltpu.async_copy(pcls_hbm.at[idxsl], gbuf_c, sem_g)
                hp = pltpu.async_copy(ppt_hbm.at[idxsl], gbuf_p, sem_g)
                hc.wait()
                hp.wait()

            clsv = jnp.where(pos, gbuf_c[...].astype(jnp.float32), 0.0)
            pptv = jnp.where(pos, gbuf_p[...], 0.0)
            pcls = clsv * cv
            isov = iso_sl[pl.ds(v * L, L)].astype(jnp.float32) * cv
            em = e0_sl[pl.ds(v * L, L)] + e1_sl[pl.ds(v * L, L)]
            pcls = jnp.where(pcls == 2.0, 0.0, pcls)
            pcls = jnp.where((pcls == 3.0) & (isov == 1.0), 1.0, pcls)
            pcls = jnp.where((pcls == 3.0) & (isov == 0.0), 2.0, pcls)
            pcls = jnp.where(pcls == 4.0, 3.0, pcls)
            em0 = em == 0.0
            pcls = jnp.where(em0 & (pcls == 1.0), 0.0, pcls)
            pcls = jnp.where(em0 & (pcls == 2.0), 0.0, pcls)
            t = jnp.clip(pcls.astype(jnp.int32), 0, 3)
            tb_sl[pl.ds(v * L, L)] = t.astype(jnp.float32)
            pm_sl[pl.ds(v * L, L)] = pptv * cv

    pltpu.sync_copy(tb_sl, outt_hbm.at[wid])
    pltpu.sync_copy(pm_sl, outpm_hbm.at[wid])


def _loss_kernel(N_F, SUBW,
                 cpred_hbm, pp_hbm, tt_hbm, pm_hbm, out_hbm,
                 cp_sl, pp_sl, t_sl, pm_sl, acc_v, sem_pre):
    NV = SUBW // L
    wid = lax.axis_index("c") * 16 + lax.axis_index("s")
    own_lo = wid * SUBW
    slice_start = pl.multiple_of(jnp.minimum(own_lo, N_F - SUBW), 16)
    own_hi = jnp.minimum(own_lo + SUBW, N_F)
    iota = lax.iota(jnp.int32, L)
    zv = jnp.zeros((L,), jnp.float32)

    pre = [
        pltpu.async_copy(cpred_hbm.at[pl.ds(4 * slice_start, 4 * SUBW)],
                         cp_sl, sem_pre),
        pltpu.async_copy(pp_hbm.at[pl.ds(slice_start, SUBW)], pp_sl, sem_pre),
        pltpu.async_copy(tt_hbm.at[wid], t_sl, sem_pre),
        pltpu.async_copy(pm_hbm.at[wid], pm_sl, sem_pre),
    ]
    for h in pre:
        h.wait()

    def node_body(v, carry):
        acc_c, acc_p = carry
        lf = v * L + iota
        f = slice_start + lf
        validn = (f >= own_lo) & (f < own_hi)
        t = t_sl[pl.ds(v * L, L)].astype(jnp.int32)

        i4 = 4 * (v * L) + 4 * iota
        l0 = plsc.load_gather(cp_sl, [i4])
        l1 = plsc.load_gather(cp_sl, [i4 + 1])
        l2 = plsc.load_gather(cp_sl, [i4 + 2])
        l3 = plsc.load_gather(cp_sl, [i4 + 3])
        m = jnp.maximum(jnp.maximum(l0, l1), jnp.maximum(l2, l3))
        ssum = (jnp.exp(l0 - m) + jnp.exp(l1 - m)
                + jnp.exp(l2 - m) + jnp.exp(l3 - m))
        tq = (ssum - 1.0) / (ssum + 1.0)
        y = tq * (2.0 + tq * tq * (2.0 / 3.0))
        y = y + ssum * jnp.exp(-y) - 1.0
        y = y + ssum * jnp.exp(-y) - 1.0
        lse = m + y
        lt = jnp.where(t == 0, l0, jnp.where(t == 1, l1,
                       jnp.where(t == 2, l2, l3)))
        wt = jnp.where(t == 0, 0.5, jnp.where(t == 1, 2.0,
                       jnp.where(t == 2, 5.0, 2.5)))
        acc_c = acc_c + jnp.where(validn, wt * (lse - lt), 0.0)
        diff = pp_sl[pl.ds(v * L, L)] - pm_sl[pl.ds(v * L, L)]
        acc_p = acc_p + jnp.where(validn, diff * diff, 0.0)
        return acc_c, acc_p

    with jax.named_scope("phase2_loss"):
        acc_c, acc_p = pl.loop(0, NV, init_carry=(zv, zv),
                               unroll=2)(node_body)

    acc_v[pl.ds(0, L)] = acc_c
    acc_v[pl.ds(L, L)] = acc_p
    pltpu.sync_copy(acc_v, out_hbm.at[wid])


def kernel(particle_pt, particle_eta, particle_phi, particle_dep_energy,
           pt_eta_phi_pred, class_pred, energy_l_0, energy_l_1,
           particle_class, particle_idx, edge_src, edge_dst,
           parent_target, isIso):
    N_P = particle_pt.shape[0]
    N_F = parent_target.shape[0]
    E = edge_src.shape[0]
    SUBW = ((N_F + NW * L - 1) // (NW * L)) * L

    mesh = plsc.VectorSubcoreMesh(core_axis_name="c", subcore_axis_name="s")
    params = pltpu.CompilerParams(needs_layout_passes=False)

    count_k = functools.partial(
        pl.kernel,
        out_type=[jax.ShapeDtypeStruct((NW, SUBW), jnp.float32),
                  jax.ShapeDtypeStruct((NW, SUBW), jnp.float32)],
        mesh=mesh,
        compiler_params=params,
        scratch_types=[
            pltpu.VMEM((SUBW,), jnp.int32),
            pltpu.VMEM((SUBW,), jnp.int32),
            pltpu.VMEM((SUBW,), jnp.float32),
            pltpu.VMEM((SUBW,), jnp.float32),
            pltpu.VMEM((L,), jnp.int32),
            pltpu.VMEM((L,), jnp.float32),
            pltpu.VMEM((SUBW,), jnp.float32),
            pltpu.VMEM((EBLK,), jnp.int32),
            pltpu.VMEM((EBLK,), jnp.int32),
            pltpu.VMEM((EBLK,), jnp.int32),
            pltpu.VMEM((EBLK,), jnp.int32),
            pltpu.VMEM((2 * L,), jnp.int32),
            pltpu.VMEM((2 * L,), jnp.float32),
            pltpu.VMEM((SUBW,), jnp.float32),
            pltpu.VMEM((SUBW,), jnp.float32),
            pltpu.VMEM((8 * L,), jnp.int32),
            pltpu.VMEM((8 * L,), jnp.int32),
            pltpu.VMEM((8 * L,), jnp.int32),
            pltpu.VMEM((8 * L,), jnp.int32),
            pltpu.SemaphoreType.DMA,
            pltpu.SemaphoreType.DMA,
            pltpu.SemaphoreType.DMA,
            pltpu.SemaphoreType.DMA,
            pltpu.SemaphoreType.DMA,
            pltpu.SemaphoreType.DMA,
        ],
    )(functools.partial(_count_kernel, N_P, N_F, E, SUBW))

    outt, outpm = count_k(particle_pt, energy_l_0, energy_l_1,
                          particle_class, edge_src, edge_dst,
                          parent_target, isIso)

    cpred_flat = class_pred.reshape(-1)
    pp0 = pt_eta_phi_pred[:, 0]

    loss_k = functools.partial(
        pl.kernel,
        out_type=jax.ShapeDtypeStruct((NW, 2 * L), jnp.float32),
        mesh=mesh,
        compiler_params=params,
        scratch_types=[
            pltpu.VMEM((4 * SUBW,), jnp.float32),
            pltpu.VMEM((SUBW,), jnp.float32),
            pltpu.VMEM((SUBW,), jnp.float32),
            pltpu.VMEM((SUBW,), jnp.float32),
            pltpu.VMEM((2 * L,), jnp.float32),
            pltpu.SemaphoreType.DMA,
        ],
    )(functools.partial(_loss_kernel, N_F, SUBW))

    out = loss_k(cpred_flat, pp0, outt, outpm)
    return (5.0 * jnp.sum(out[:, :L]) + jnp.sum(out[:, L:])) / N_F
